# Initial kernel scaffold; baseline (speedup 1.0000x reference)
import jax
import jax.numpy as jnp
from jax.experimental import pallas as pl


def kernel(features, centroids):
    raise NotImplementedError("write your pallas kernel here")


# speedup vs baseline: 3.2895x; 3.2895x over previous
import functools

import jax
import jax.numpy as jnp
from jax.experimental import pallas as pl
from jax.experimental.pallas import tpu as pltpu

_N = 65536
_D = 512
_K = 1024
_INV_T = 2.0


def _loss_kernel(feat_ref, cent_ref, out_ref, cnorm_ref, s_ref, acc_ref, *, blk):
    i = pl.program_id(0)

    @pl.when(i == 0)
    def _prep():
        c = cent_ref[...]
        norm = jnp.sqrt(jnp.sum(c * c, axis=1, keepdims=True))
        cn = c / jnp.maximum(norm, 1e-12)
        cnorm_ref[...] = cn
        g = jax.lax.dot_general(
            cn, cn, (((1,), (1,)), ((), ())),
            preferred_element_type=jnp.float32,
        )
        s_ref[...] = jnp.sum(jnp.exp(g * _INV_T), axis=0, keepdims=True)
        acc_ref[...] = jnp.zeros((1, 1), jnp.float32)

    f = feat_ref[...]
    cn = cnorm_ref[...]
    prod = jax.lax.dot_general(
        f, cn, (((1,), (1,)), ((), ())),
        preferred_element_type=jnp.float32,
    ) * _INV_T
    m = jnp.max(prod, axis=1, keepdims=True)
    iota = jax.lax.broadcasted_iota(jnp.int32, (blk, _K), 1)
    masked = jnp.where(prod == m, iota, _K)
    idx = jnp.min(masked, axis=1, keepdims=True)
    s_pick = jnp.sum(
        jnp.where(iota == idx, s_ref[...], 0.0), axis=1, keepdims=True
    )
    term = m - jnp.log(jnp.exp(m) + s_pick)
    acc_ref[...] += jnp.sum(term, axis=0, keepdims=True).reshape(1, 1)

    @pl.when(i == pl.num_programs(0) - 1)
    def _fin():
        out_ref[...] = -acc_ref[...] / _N


@functools.partial(jax.jit, static_argnames=("blk",))
def _run(features, centroids, blk=1024):
    out = pl.pallas_call(
        functools.partial(_loss_kernel, blk=blk),
        grid=(_N // blk,),
        in_specs=[
            pl.BlockSpec((blk, _D), lambda i: (i, 0)),
            pl.BlockSpec((_K, _D), lambda i: (0, 0)),
        ],
        out_specs=pl.BlockSpec((1, 1), lambda i: (0, 0)),
        out_shape=jax.ShapeDtypeStruct((1, 1), jnp.float32),
        scratch_shapes=[
            pltpu.VMEM((_K, _D), jnp.float32),
            pltpu.VMEM((1, _K), jnp.float32),
            pltpu.VMEM((1, 1), jnp.float32),
        ],
    )(features, centroids)
    return out[0, 0]


def kernel(features, centroids):
    return _run(features, centroids)
